# baseline (speedup 1.0000x reference)
import functools

import jax
import jax.numpy as jnp
from jax import lax
from jax.experimental import pallas as pl
from jax.experimental.pallas import tpu as pltpu
from jax.experimental.pallas import tpu_sc as plsc

_N = 10000
_E = 160000
_D = 256
_DE = 16
_G = 64

_NP = 10240
_NW = 32
_CH = 64
_NCH = 40
_EPT = _NCH * _CH
_ESUB = _NW * _EPT
_EP = 2 * _ESUB
_RPT = _NP // 16
_EBLK = 1280
_NBLK = 200


def _edge_mlp_body(ea_ref, w_ref, b_ref, e0_ref, e1_ref):
    ea = ea_ref[...]
    b = b_ref[...]
    w = w_ref[...]
    g = _EBLK // 8
    for r in range(8):
        z = jnp.dot(ea[:, r * _DE:(r + 1) * _DE], w,
                    preferred_element_type=jnp.float32) + b
        z = z * jax.nn.sigmoid(z)
        e0_ref[pl.ds(r * g, g), :] = z[:, :128]
        e1_ref[pl.ds(r * g, g), :] = z[:, 128:]


def _edge_mlp(ea128, eW, eb_row, t):
    blk = _EBLK
    rblk = _EBLK * _DE // 128
    grid = _ESUB // blk
    base = t * grid
    last = _E // blk - 1
    return pl.pallas_call(
        _edge_mlp_body,
        grid=(grid,),
        in_specs=[
            pl.BlockSpec((rblk, 128), lambda i: (jnp.minimum(base + i, last), 0)),
            pl.BlockSpec((_DE, _D), lambda i: (0, 0)),
            pl.BlockSpec((1, _D), lambda i: (0, 0)),
        ],
        out_specs=[
            pl.BlockSpec((blk, 128), lambda i: (i, 0)),
            pl.BlockSpec((blk, 128), lambda i: (i, 0)),
        ],
        out_shape=[jax.ShapeDtypeStruct((_ESUB, 128), jnp.float32)] * 2,
    )(ea128, eW, eb_row)


def _sc_edge_body(x0, x1, e0, e1, srcs, dsts, out,
                  i_v0, i_v1, i_v2, i_v3, xs_v0, xs_v1, e_v0, e_v1, acc,
                  gsem0, gsem1, esem0, esem1,
                  isem0, isem1, isem2, isem3):
    cid = lax.axis_index("c")
    sid = lax.axis_index("s")
    wid = sid * 2 + cid

    i_b = (i_v0, i_v1, i_v2, i_v3)
    isems = (isem0, isem1, isem2, isem3)
    xs_b = (xs_v0, xs_v1)
    e_b = (e_v0, e_v1)
    gsems = (gsem0, gsem1)
    esems = (esem0, esem1)

    def _idx_load(q, s):
        pltpu.async_copy(srcs.at[wid, q], i_b[s].at[0], isems[s])
        pltpu.async_copy(dsts.at[wid, q], i_b[s].at[1], isems[s])

    def _idx_wait(s):
        pltpu.make_async_copy(srcs.at[0, pl.ds(0, 2)], i_b[s],
                              isems[s]).wait()

    for h, (xh, eh) in enumerate(((x0, e0), (x1, e1))):

        def _issue(q, s, p):
            pltpu.async_copy(xh.at[i_b[s].at[0]], xs_b[p], gsems[p])
            pltpu.async_copy(eh.at[pl.ds(wid * _EPT + q * _CH, _CH)],
                             e_b[p], esems[p])

        def _wait(p):
            pltpu.make_async_copy(xh.at[pl.ds(0, _CH)], xs_b[p],
                                  gsems[p]).wait()
            pltpu.make_async_copy(eh.at[pl.ds(0, _CH)], e_b[p],
                                  esems[p]).wait()

        @pl.loop(0, _CH)
        def _zrow(i):
            for j in range(8):
                xs_v0[i, pl.ds(j * 16, 16)] = jnp.zeros((16,), jnp.float32)

        @pl.loop(0, _RPT // _CH)
        def _zero(k):
            pltpu.sync_copy(xs_v0, acc.at[pl.ds(sid * _RPT + k * _CH, _CH)])

        plsc.subcore_barrier()

        _idx_load(0, 0)
        _idx_load(1, 1)
        _idx_wait(0)
        _issue(0, 0, 0)

        @pl.loop(0, _NCH // 4)
        def _quad(g):
            for k in range(4):
                q = g * 4 + k
                p = k % 2
                s1 = (k + 1) % 4
                s2 = (k + 2) % 4
                qn = jnp.minimum(q + 1, _NCH - 1)
                qn2 = jnp.minimum(q + 2, _NCH - 1)
                _idx_wait(s1)
                _issue(qn, s1, 1 - p)
                _idx_load(qn2, s2)
                _wait(p)

                @pl.loop(0, _CH)
                def _row(i):
                    for j in range(8):
                        v = (xs_b[p][i, pl.ds(j * 16, 16)]
                             + e_b[p][i, pl.ds(j * 16, 16)])
                        xs_b[p][i, pl.ds(j * 16, 16)] = jnp.maximum(v, 0.0)

                pltpu.sync_copy(xs_b[p], acc.at[i_b[k].at[1]], add=True)

        _wait(0)
        _idx_wait(1)

        plsc.subcore_barrier()
        pltpu.sync_copy(acc.at[pl.ds(sid * _RPT, _RPT)],
                        out.at[cid, h, pl.ds(sid * _RPT, _RPT)])
        plsc.subcore_barrier()


def _sc_edge(x0, x1, e0, e1, srcs, dsts):
    mesh = plsc.VectorSubcoreMesh(core_axis_name="c", subcore_axis_name="s")
    fn = pl.kernel(
        _sc_edge_body,
        out_type=jax.ShapeDtypeStruct((2, 2, _NP, 128), jnp.float32),
        mesh=mesh,
        scratch_types=(
            [pltpu.VMEM((2, _CH), jnp.int32)] * 4
            + [pltpu.VMEM((_CH, 128), jnp.float32)] * 4
            + [pltpu.VMEM_SHARED((_NP, 128), jnp.float32)]
            + [pltpu.SemaphoreType.DMA] * 8
        ),
    )
    return fn(x0, x1, e0, e1, srcs, dsts)


def _node_mlp_body(p00, p01, p10, p11, q00, q01, q10, q11,
                   x_ref, n2g_ref, w1, b1, w2, b2,
                   eps_ref, h_ref, st_ref):
    i = pl.program_id(0)
    agg0 = (p00[0, 0] + p10[0, 0]) + (q00[0, 0] + q10[0, 0])
    agg1 = (p01[0, 0] + p11[0, 0]) + (q01[0, 0] + q11[0, 0])
    agg = jnp.concatenate([agg0, agg1], axis=1)
    hb = agg + (1.0 + eps_ref[0]) * x_ref[...]
    t = jnp.dot(hb, w1[...], preferred_element_type=jnp.float32) + b1[...]
    t = t * jax.nn.sigmoid(t)
    t = jnp.dot(t, w2[...], preferred_element_type=jnp.float32) + b2[...]
    t = t * jax.nn.sigmoid(t)
    h_ref[...] = t

    n2g = n2g_ref[0, 0, :]
    ohg = (lax.broadcasted_iota(jnp.int32, (128, _NBLK), 0)
           == n2g[None, :]).astype(jnp.float32)
    ones_d = jnp.ones((_D, 1), jnp.float32)
    ones_n = jnp.ones((_NBLK, 1), jnp.float32)
    rs = jnp.dot(t, ones_d, preferred_element_type=jnp.float32)
    rs2 = jnp.dot(t * t, ones_d, preferred_element_type=jnp.float32)
    cnt = jnp.dot(ohg, ones_n, preferred_element_type=jnp.float32)
    s1 = jnp.dot(ohg, rs, preferred_element_type=jnp.float32)
    s2 = jnp.dot(ohg, rs2, preferred_element_type=jnp.float32)
    contrib = jnp.concatenate(
        [cnt, s1, s2, jnp.zeros((128, 125), jnp.float32)], axis=1)

    @pl.when(i == 0)
    def _():
        st_ref[...] = jnp.zeros_like(st_ref)

    st_ref[...] += contrib


def _node_mlp(parts_a, parts_b, x, n2g3, w1, b1_row, w2, b2_row, eps):
    grid = _N // _NBLK
    pspec = [
        pl.BlockSpec((1, 1, _NBLK, 128), lambda i: (0, 0, i, 0)),
        pl.BlockSpec((1, 1, _NBLK, 128), lambda i: (0, 1, i, 0)),
        pl.BlockSpec((1, 1, _NBLK, 128), lambda i: (1, 0, i, 0)),
        pl.BlockSpec((1, 1, _NBLK, 128), lambda i: (1, 1, i, 0)),
    ]
    return pl.pallas_call(
        _node_mlp_body,
        grid=(grid,),
        in_specs=pspec + pspec + [
            pl.BlockSpec((_NBLK, _D), lambda i: (i, 0)),
            pl.BlockSpec((1, 1, _NBLK), lambda i: (i, 0, 0)),
            pl.BlockSpec((_D, _D), lambda i: (0, 0)),
            pl.BlockSpec((1, _D), lambda i: (0, 0)),
            pl.BlockSpec((_D, _D), lambda i: (0, 0)),
            pl.BlockSpec((1, _D), lambda i: (0, 0)),
            pl.BlockSpec(memory_space=pltpu.SMEM),
        ],
        out_specs=[
            pl.BlockSpec((_NBLK, _D), lambda i: (i, 0)),
            pl.BlockSpec((128, 128), lambda i: (0, 0)),
        ],
        out_shape=[
            jax.ShapeDtypeStruct((_N, _D), jnp.float32),
            jax.ShapeDtypeStruct((128, 128), jnp.float32),
        ],
    )(parts_a, parts_a, parts_a, parts_a, parts_b, parts_b, parts_b,
      parts_b, x, n2g3, w1, b1_row, w2, b2_row, eps)


def _final_body(h_ref, x_ref, st_ref, n2g_ref, w_ref, b_ref, o_ref):
    st = st_ref[...]
    cnt = st[:, 0:1]
    s1 = st[:, 1:2]
    s2 = st[:, 2:3]
    norm = jnp.maximum(cnt, 1.0) * float(_D)
    mean = s1 / norm
    var = s2 / norm - mean * mean
    inv = lax.rsqrt(var + 1e-5)

    n2g = n2g_ref[0, 0, :]
    ohn = (n2g[:, None]
           == lax.broadcasted_iota(jnp.int32, (_NBLK, 128), 1)).astype(jnp.float32)
    rmean = jnp.dot(ohn, mean, preferred_element_type=jnp.float32)
    rinv = jnp.dot(ohn, inv, preferred_element_type=jnp.float32)

    t = (h_ref[...] - rmean) * rinv * w_ref[...] + b_ref[...]
    o_ref[...] = jnp.maximum((t + x_ref[...]) * 0.5, 0.0)


def _final(h, x, st, n2g3, w_row, b_row):
    grid = _N // _NBLK
    return pl.pallas_call(
        _final_body,
        grid=(grid,),
        in_specs=[
            pl.BlockSpec((_NBLK, _D), lambda i: (i, 0)),
            pl.BlockSpec((_NBLK, _D), lambda i: (i, 0)),
            pl.BlockSpec((128, 128), lambda i: (0, 0)),
            pl.BlockSpec((1, 1, _NBLK), lambda i: (i, 0, 0)),
            pl.BlockSpec((1, _D), lambda i: (0, 0)),
            pl.BlockSpec((1, _D), lambda i: (0, 0)),
        ],
        out_specs=pl.BlockSpec((_NBLK, _D), lambda i: (i, 0)),
        out_shape=jax.ShapeDtypeStruct((_N, _D), jnp.float32),
    )(h, x, st, n2g3, w_row, b_row)


def kernel(x, edge_index, edge_attr, node2graph,
           c1_edge_W, c1_edge_b, c1_W1, c1_b1, c1_W2, c1_b2, c1_eps,
           c2_edge_W, c2_edge_b, c2_W1, c2_b1, c2_W2, c2_b2, c2_eps,
           gn1_w, gn1_b, gn2_w, gn2_b):
    src = edge_index[0].astype(jnp.int32)
    dst = edge_index[1].astype(jnp.int32)

    pad_e = _EP - _E
    psrc = jnp.arange(pad_e, dtype=jnp.int32) % _N
    src_p = jnp.concatenate([src, psrc])
    trash = _N + (jnp.arange(pad_e, dtype=jnp.int32) % (_NP - _N))
    dst_p = jnp.concatenate([dst, trash])
    def _perm(a):
        a = a.reshape(_EP // _EBLK, _EBLK // 8, 8)
        return a.transpose(0, 2, 1).reshape(2, _NW, _NCH, _CH)

    srcs = _perm(src_p)
    dsts = _perm(dst_p)

    x0 = x[:, :128]
    x1 = x[:, 128:]

    eb_row = c2_edge_b.reshape(1, _D)
    ea128 = edge_attr.reshape(_E * _DE // 128, 128)
    ea0, ea1 = _edge_mlp(ea128, c2_edge_W, eb_row, 0)
    eb0, eb1 = _edge_mlp(ea128, c2_edge_W, eb_row, 1)
    parts_a = _sc_edge(x0, x1, ea0, ea1, srcs[0], dsts[0])
    parts_b = _sc_edge(x0, x1, eb0, eb1, srcs[1], dsts[1])

    n2g3 = node2graph.astype(jnp.int32).reshape(_N // _NBLK, 1, _NBLK)

    h, st = _node_mlp(parts_a, parts_b, x, n2g3, c2_W1,
                      c2_b1.reshape(1, _D), c2_W2, c2_b2.reshape(1, _D),
                      c2_eps)
    out = _final(h, x, st, n2g3, gn2_w.reshape(1, _D),
                 gn2_b.reshape(1, _D))
    return out


# speedup vs baseline: 1.1439x; 1.1439x over previous
import functools

import jax
import jax.numpy as jnp
from jax import lax
from jax.experimental import pallas as pl
from jax.experimental.pallas import tpu as pltpu
from jax.experimental.pallas import tpu_sc as plsc

_N = 10000
_E = 160000
_D = 256
_DE = 16
_G = 64

_NP = 10240
_NW = 32
_CH = 64
_NCH = 40
_EPT = _NCH * _CH
_ESUB = _NW * _EPT
_EP = 2 * _ESUB
_RPT = _NP // 16
_EBLK = 1280
_NBLK = 400


def _edge_mlp_body(ea_ref, w_ref, b_ref, e0_ref, e1_ref):
    e = jnp.dot(ea_ref[...], w_ref[...], preferred_element_type=jnp.float32)
    e = e + b_ref[...]
    e = e * jax.nn.sigmoid(e)
    e0_ref[...] = e[:, :128]
    e1_ref[...] = e[:, 128:]


def _edge_mlp(ea, eW, eb_row, t):
    blk = _EBLK
    grid = _ESUB // blk
    base = t * grid
    last = _E // blk - 1
    return pl.pallas_call(
        _edge_mlp_body,
        grid=(grid,),
        in_specs=[
            pl.BlockSpec((blk, _DE), lambda i: (jnp.minimum(base + i, last), 0)),
            pl.BlockSpec((_DE, _D), lambda i: (0, 0)),
            pl.BlockSpec((1, _D), lambda i: (0, 0)),
        ],
        out_specs=[
            pl.BlockSpec((blk, 128), lambda i: (i, 0)),
            pl.BlockSpec((blk, 128), lambda i: (i, 0)),
        ],
        out_shape=[jax.ShapeDtypeStruct((_ESUB, 128), jnp.float32)] * 2,
    )(ea, eW, eb_row)


def _sc_edge_body(x0, x1, e0, e1, srcs, dsts, out,
                  i_v0, i_v1, i_v2, i_v3, xs_v0, xs_v1, e_v0, e_v1, acc,
                  gsem0, gsem1, esem0, esem1,
                  isem0, isem1, isem2, isem3):
    cid = lax.axis_index("c")
    sid = lax.axis_index("s")
    wid = sid * 2 + cid

    i_b = (i_v0, i_v1, i_v2, i_v3)
    isems = (isem0, isem1, isem2, isem3)
    xs_b = (xs_v0, xs_v1)
    e_b = (e_v0, e_v1)
    gsems = (gsem0, gsem1)
    esems = (esem0, esem1)

    def _idx_load(q, s):
        pltpu.async_copy(srcs.at[wid, q], i_b[s].at[0], isems[s])
        pltpu.async_copy(dsts.at[wid, q], i_b[s].at[1], isems[s])

    def _idx_wait(s):
        pltpu.make_async_copy(srcs.at[0, pl.ds(0, 2)], i_b[s],
                              isems[s]).wait()

    for h, (xh, eh) in enumerate(((x0, e0), (x1, e1))):

        def _issue(q, s, p):
            pltpu.async_copy(xh.at[i_b[s].at[0]], xs_b[p], gsems[p])
            pltpu.async_copy(eh.at[pl.ds(wid * _EPT + q * _CH, _CH)],
                             e_b[p], esems[p])

        def _wait(p):
            pltpu.make_async_copy(xh.at[pl.ds(0, _CH)], xs_b[p],
                                  gsems[p]).wait()
            pltpu.make_async_copy(eh.at[pl.ds(0, _CH)], e_b[p],
                                  esems[p]).wait()

        @pl.loop(0, _CH)
        def _zrow(i):
            for j in range(8):
                xs_v0[i, pl.ds(j * 16, 16)] = jnp.zeros((16,), jnp.float32)

        @pl.loop(0, _RPT // _CH)
        def _zero(k):
            pltpu.sync_copy(xs_v0, acc.at[pl.ds(sid * _RPT + k * _CH, _CH)])

        plsc.subcore_barrier()

        _idx_load(0, 0)
        _idx_load(1, 1)
        _idx_wait(0)
        _issue(0, 0, 0)

        @pl.loop(0, _NCH // 4)
        def _quad(g):
            for k in range(4):
                q = g * 4 + k
                p = k % 2
                s1 = (k + 1) % 4
                s2 = (k + 2) % 4
                qn = jnp.minimum(q + 1, _NCH - 1)
                qn2 = jnp.minimum(q + 2, _NCH - 1)
                _idx_wait(s1)
                _issue(qn, s1, 1 - p)
                _idx_load(qn2, s2)
                _wait(p)

                @pl.loop(0, _CH)
                def _row(i):
                    for j in range(8):
                        v = (xs_b[p][i, pl.ds(j * 16, 16)]
                             + e_b[p][i, pl.ds(j * 16, 16)])
                        xs_b[p][i, pl.ds(j * 16, 16)] = jnp.maximum(v, 0.0)

                pltpu.sync_copy(xs_b[p], acc.at[i_b[k].at[1]], add=True)

        _wait(0)
        _idx_wait(1)

        plsc.subcore_barrier()
        pltpu.sync_copy(acc.at[pl.ds(sid * _RPT, _RPT)],
                        out.at[cid, h, pl.ds(sid * _RPT, _RPT)])
        plsc.subcore_barrier()


def _sc_edge(x0, x1, e0, e1, srcs, dsts):
    mesh = plsc.VectorSubcoreMesh(core_axis_name="c", subcore_axis_name="s")
    fn = pl.kernel(
        _sc_edge_body,
        out_type=jax.ShapeDtypeStruct((2, 2, _NP, 128), jnp.float32),
        mesh=mesh,
        scratch_types=(
            [pltpu.VMEM((2, _CH), jnp.int32)] * 4
            + [pltpu.VMEM((_CH, 128), jnp.float32)] * 4
            + [pltpu.VMEM_SHARED((_NP, 128), jnp.float32)]
            + [pltpu.SemaphoreType.DMA] * 8
        ),
    )
    return fn(x0, x1, e0, e1, srcs, dsts)


def _node_mlp_body(p00, p01, p10, p11, q00, q01, q10, q11,
                   x_ref, n2g_ref, w1, b1, w2, b2,
                   eps_ref, h_ref, st_ref):
    i = pl.program_id(0)
    agg0 = (p00[0, 0] + p10[0, 0]) + (q00[0, 0] + q10[0, 0])
    agg1 = (p01[0, 0] + p11[0, 0]) + (q01[0, 0] + q11[0, 0])
    agg = jnp.concatenate([agg0, agg1], axis=1)
    hb = agg + (1.0 + eps_ref[0]) * x_ref[...]
    t = jnp.dot(hb, w1[...], preferred_element_type=jnp.float32) + b1[...]
    t = t * jax.nn.sigmoid(t)
    t = jnp.dot(t, w2[...], preferred_element_type=jnp.float32) + b2[...]
    t = t * jax.nn.sigmoid(t)
    h_ref[...] = t

    n2g = n2g_ref[0, 0, :]
    ohg = (lax.broadcasted_iota(jnp.int32, (128, _NBLK), 0)
           == n2g[None, :]).astype(jnp.float32)
    ones_d = jnp.ones((_D, 1), jnp.float32)
    ones_n = jnp.ones((_NBLK, 1), jnp.float32)
    rs = jnp.dot(t, ones_d, preferred_element_type=jnp.float32)
    rs2 = jnp.dot(t * t, ones_d, preferred_element_type=jnp.float32)
    cnt = jnp.dot(ohg, ones_n, preferred_element_type=jnp.float32)
    s1 = jnp.dot(ohg, rs, preferred_element_type=jnp.float32)
    s2 = jnp.dot(ohg, rs2, preferred_element_type=jnp.float32)
    contrib = jnp.concatenate(
        [cnt, s1, s2, jnp.zeros((128, 125), jnp.float32)], axis=1)

    @pl.when(i == 0)
    def _():
        st_ref[...] = jnp.zeros_like(st_ref)

    st_ref[...] += contrib


def _node_mlp(parts_a, parts_b, x, n2g3, w1, b1_row, w2, b2_row, eps):
    grid = _N // _NBLK
    pspec = [
        pl.BlockSpec((1, 1, _NBLK, 128), lambda i: (0, 0, i, 0)),
        pl.BlockSpec((1, 1, _NBLK, 128), lambda i: (0, 1, i, 0)),
        pl.BlockSpec((1, 1, _NBLK, 128), lambda i: (1, 0, i, 0)),
        pl.BlockSpec((1, 1, _NBLK, 128), lambda i: (1, 1, i, 0)),
    ]
    return pl.pallas_call(
        _node_mlp_body,
        grid=(grid,),
        in_specs=pspec + pspec + [
            pl.BlockSpec((_NBLK, _D), lambda i: (i, 0)),
            pl.BlockSpec((1, 1, _NBLK), lambda i: (i, 0, 0)),
            pl.BlockSpec((_D, _D), lambda i: (0, 0)),
            pl.BlockSpec((1, _D), lambda i: (0, 0)),
            pl.BlockSpec((_D, _D), lambda i: (0, 0)),
            pl.BlockSpec((1, _D), lambda i: (0, 0)),
            pl.BlockSpec(memory_space=pltpu.SMEM),
        ],
        out_specs=[
            pl.BlockSpec((_NBLK, _D), lambda i: (i, 0)),
            pl.BlockSpec((128, 128), lambda i: (0, 0)),
        ],
        out_shape=[
            jax.ShapeDtypeStruct((_N, _D), jnp.float32),
            jax.ShapeDtypeStruct((128, 128), jnp.float32),
        ],
    )(parts_a, parts_a, parts_a, parts_a, parts_b, parts_b, parts_b,
      parts_b, x, n2g3, w1, b1_row, w2, b2_row, eps)


def _final_body(h_ref, x_ref, st_ref, n2g_ref, w_ref, b_ref, o_ref):
    st = st_ref[...]
    cnt = st[:, 0:1]
    s1 = st[:, 1:2]
    s2 = st[:, 2:3]
    norm = jnp.maximum(cnt, 1.0) * float(_D)
    mean = s1 / norm
    var = s2 / norm - mean * mean
    inv = lax.rsqrt(var + 1e-5)

    n2g = n2g_ref[0, 0, :]
    ohn = (n2g[:, None]
           == lax.broadcasted_iota(jnp.int32, (_NBLK, 128), 1)).astype(jnp.float32)
    rmean = jnp.dot(ohn, mean, preferred_element_type=jnp.float32)
    rinv = jnp.dot(ohn, inv, preferred_element_type=jnp.float32)

    t = (h_ref[...] - rmean) * rinv * w_ref[...] + b_ref[...]
    o_ref[...] = jnp.maximum((t + x_ref[...]) * 0.5, 0.0)


def _final(h, x, st, n2g3, w_row, b_row):
    grid = _N // _NBLK
    return pl.pallas_call(
        _final_body,
        grid=(grid,),
        in_specs=[
            pl.BlockSpec((_NBLK, _D), lambda i: (i, 0)),
            pl.BlockSpec((_NBLK, _D), lambda i: (i, 0)),
            pl.BlockSpec((128, 128), lambda i: (0, 0)),
            pl.BlockSpec((1, 1, _NBLK), lambda i: (i, 0, 0)),
            pl.BlockSpec((1, _D), lambda i: (0, 0)),
            pl.BlockSpec((1, _D), lambda i: (0, 0)),
        ],
        out_specs=pl.BlockSpec((_NBLK, _D), lambda i: (i, 0)),
        out_shape=jax.ShapeDtypeStruct((_N, _D), jnp.float32),
    )(h, x, st, n2g3, w_row, b_row)


def kernel(x, edge_index, edge_attr, node2graph,
           c1_edge_W, c1_edge_b, c1_W1, c1_b1, c1_W2, c1_b2, c1_eps,
           c2_edge_W, c2_edge_b, c2_W1, c2_b1, c2_W2, c2_b2, c2_eps,
           gn1_w, gn1_b, gn2_w, gn2_b):
    src = edge_index[0].astype(jnp.int32)
    dst = edge_index[1].astype(jnp.int32)

    pad_e = _EP - _E
    psrc = jnp.arange(pad_e, dtype=jnp.int32) % _N
    src_p = jnp.concatenate([src, psrc])
    trash = _N + (jnp.arange(pad_e, dtype=jnp.int32) % (_NP - _N))
    dst_p = jnp.concatenate([dst, trash])
    srcs = src_p.reshape(2, _NW, _NCH, _CH)
    dsts = dst_p.reshape(2, _NW, _NCH, _CH)

    x0 = x[:, :128]
    x1 = x[:, 128:]

    eb_row = c2_edge_b.reshape(1, _D)
    ea0, ea1 = _edge_mlp(edge_attr, c2_edge_W, eb_row, 0)
    eb0, eb1 = _edge_mlp(edge_attr, c2_edge_W, eb_row, 1)
    parts_a = _sc_edge(x0, x1, ea0, ea1, srcs[0], dsts[0])
    parts_b = _sc_edge(x0, x1, eb0, eb1, srcs[1], dsts[1])

    n2g3 = node2graph.astype(jnp.int32).reshape(_N // _NBLK, 1, _NBLK)

    h, st = _node_mlp(parts_a, parts_b, x, n2g3, c2_W1,
                      c2_b1.reshape(1, _D), c2_W2, c2_b2.reshape(1, _D),
                      c2_eps)
    out = _final(h, x, st, n2g3, gn2_w.reshape(1, _D),
                 gn2_b.reshape(1, _D))
    return out


# speedup vs baseline: 1.1949x; 1.0445x over previous
import functools

import jax
import jax.numpy as jnp
from jax import lax
from jax.experimental import pallas as pl
from jax.experimental.pallas import tpu as pltpu
from jax.experimental.pallas import tpu_sc as plsc

_N = 10000
_E = 160000
_D = 256
_DE = 16
_G = 64

_NP = 10240
_NW = 32
_CH = 64
_NCH = 40
_EPT = _NCH * _CH
_ESUB = _NW * _EPT
_EP = 2 * _ESUB
_RPT = _NP // 16
_EBLK = 1280
_NBLK = 1000


def _edge_mlp_body(ea_ref, w_ref, b_ref, e0_ref, e1_ref):
    e = jnp.dot(ea_ref[...], w_ref[...], preferred_element_type=jnp.float32)
    e = e + b_ref[...]
    e = e * jax.nn.sigmoid(e)
    e0_ref[...] = e[:, :128]
    e1_ref[...] = e[:, 128:]


def _edge_mlp(ea, eW, eb_row, t):
    blk = _EBLK
    grid = _ESUB // blk
    base = t * grid
    last = _E // blk - 1
    return pl.pallas_call(
        _edge_mlp_body,
        grid=(grid,),
        in_specs=[
            pl.BlockSpec((blk, _DE), lambda i: (jnp.minimum(base + i, last), 0)),
            pl.BlockSpec((_DE, _D), lambda i: (0, 0)),
            pl.BlockSpec((1, _D), lambda i: (0, 0)),
        ],
        out_specs=[
            pl.BlockSpec((blk, 128), lambda i: (i, 0)),
            pl.BlockSpec((blk, 128), lambda i: (i, 0)),
        ],
        out_shape=[jax.ShapeDtypeStruct((_ESUB, 128), jnp.float32)] * 2,
    )(ea, eW, eb_row)


def _sc_edge_body(x0, x1, e0, e1, srcs, dsts, out,
                  i_v0, i_v1, i_v2, i_v3, xs_v0, xs_v1, e_v0, e_v1, acc,
                  gsem0, gsem1, esem0, esem1,
                  isem0, isem1, isem2, isem3):
    cid = lax.axis_index("c")
    sid = lax.axis_index("s")
    wid = sid * 2 + cid

    i_b = (i_v0, i_v1, i_v2, i_v3)
    isems = (isem0, isem1, isem2, isem3)
    xs_b = (xs_v0, xs_v1)
    e_b = (e_v0, e_v1)
    gsems = (gsem0, gsem1)
    esems = (esem0, esem1)

    def _idx_load(q, s):
        pltpu.async_copy(srcs.at[wid, q], i_b[s].at[0], isems[s])
        pltpu.async_copy(dsts.at[wid, q], i_b[s].at[1], isems[s])

    def _idx_wait(s):
        pltpu.make_async_copy(srcs.at[0, pl.ds(0, 2)], i_b[s],
                              isems[s]).wait()

    for h, (xh, eh) in enumerate(((x0, e0), (x1, e1))):

        def _issue(q, s, p):
            pltpu.async_copy(xh.at[i_b[s].at[0]], xs_b[p], gsems[p])
            pltpu.async_copy(eh.at[pl.ds(wid * _EPT + q * _CH, _CH)],
                             e_b[p], esems[p])

        def _wait(p):
            pltpu.make_async_copy(xh.at[pl.ds(0, _CH)], xs_b[p],
                                  gsems[p]).wait()
            pltpu.make_async_copy(eh.at[pl.ds(0, _CH)], e_b[p],
                                  esems[p]).wait()

        @pl.loop(0, _CH)
        def _zrow(i):
            for j in range(8):
                xs_v0[i, pl.ds(j * 16, 16)] = jnp.zeros((16,), jnp.float32)

        @pl.loop(0, _RPT // _CH)
        def _zero(k):
            pltpu.sync_copy(xs_v0, acc.at[pl.ds(sid * _RPT + k * _CH, _CH)])

        plsc.subcore_barrier()

        _idx_load(0, 0)
        _idx_load(1, 1)
        _idx_wait(0)
        _issue(0, 0, 0)

        @pl.loop(0, _NCH // 4)
        def _quad(g):
            for k in range(4):
                q = g * 4 + k
                p = k % 2
                s1 = (k + 1) % 4
                s2 = (k + 2) % 4
                qn = jnp.minimum(q + 1, _NCH - 1)
                qn2 = jnp.minimum(q + 2, _NCH - 1)
                _idx_wait(s1)
                _issue(qn, s1, 1 - p)
                _idx_load(qn2, s2)
                _wait(p)

                @pl.loop(0, _CH)
                def _row(i):
                    for j in range(8):
                        v = (xs_b[p][i, pl.ds(j * 16, 16)]
                             + e_b[p][i, pl.ds(j * 16, 16)])
                        xs_b[p][i, pl.ds(j * 16, 16)] = jnp.maximum(v, 0.0)

                pltpu.sync_copy(xs_b[p], acc.at[i_b[k].at[1]], add=True)

        _wait(0)
        _idx_wait(1)

        plsc.subcore_barrier()
        pltpu.sync_copy(acc.at[pl.ds(sid * _RPT, _RPT)],
                        out.at[cid, h, pl.ds(sid * _RPT, _RPT)])
        plsc.subcore_barrier()


def _sc_edge(x0, x1, e0, e1, srcs, dsts):
    mesh = plsc.VectorSubcoreMesh(core_axis_name="c", subcore_axis_name="s")
    fn = pl.kernel(
        _sc_edge_body,
        out_type=jax.ShapeDtypeStruct((2, 2, _NP, 128), jnp.float32),
        mesh=mesh,
        scratch_types=(
            [pltpu.VMEM((2, _CH), jnp.int32)] * 4
            + [pltpu.VMEM((_CH, 128), jnp.float32)] * 4
            + [pltpu.VMEM_SHARED((_NP, 128), jnp.float32)]
            + [pltpu.SemaphoreType.DMA] * 8
        ),
    )
    return fn(x0, x1, e0, e1, srcs, dsts)


def _node_mlp_body(p00, p01, p10, p11, q00, q01, q10, q11,
                   x_ref, n2g_ref, w1, b1, w2, b2,
                   eps_ref, h_ref, st_ref):
    i = pl.program_id(0)
    agg0 = (p00[0, 0] + p10[0, 0]) + (q00[0, 0] + q10[0, 0])
    agg1 = (p01[0, 0] + p11[0, 0]) + (q01[0, 0] + q11[0, 0])
    agg = jnp.concatenate([agg0, agg1], axis=1)
    hb = agg + (1.0 + eps_ref[0]) * x_ref[...]
    t = jnp.dot(hb, w1[...], preferred_element_type=jnp.float32) + b1[...]
    t = t * jax.nn.sigmoid(t)
    t = jnp.dot(t, w2[...], preferred_element_type=jnp.float32) + b2[...]
    t = t * jax.nn.sigmoid(t)
    h_ref[...] = t

    n2g = n2g_ref[0, 0, :]
    ohg = (lax.broadcasted_iota(jnp.int32, (128, _NBLK), 0)
           == n2g[None, :]).astype(jnp.float32)
    ones_d = jnp.ones((_D, 1), jnp.float32)
    ones_n = jnp.ones((_NBLK, 1), jnp.float32)
    rs = jnp.dot(t, ones_d, preferred_element_type=jnp.float32)
    rs2 = jnp.dot(t * t, ones_d, preferred_element_type=jnp.float32)
    cnt = jnp.dot(ohg, ones_n, preferred_element_type=jnp.float32)
    s1 = jnp.dot(ohg, rs, preferred_element_type=jnp.float32)
    s2 = jnp.dot(ohg, rs2, preferred_element_type=jnp.float32)
    contrib = jnp.concatenate(
        [cnt, s1, s2, jnp.zeros((128, 125), jnp.float32)], axis=1)

    @pl.when(i == 0)
    def _():
        st_ref[...] = jnp.zeros_like(st_ref)

    st_ref[...] += contrib


def _node_mlp(parts_a, parts_b, x, n2g3, w1, b1_row, w2, b2_row, eps):
    grid = _N // _NBLK
    pspec = [
        pl.BlockSpec((1, 1, _NBLK, 128), lambda i: (0, 0, i, 0)),
        pl.BlockSpec((1, 1, _NBLK, 128), lambda i: (0, 1, i, 0)),
        pl.BlockSpec((1, 1, _NBLK, 128), lambda i: (1, 0, i, 0)),
        pl.BlockSpec((1, 1, _NBLK, 128), lambda i: (1, 1, i, 0)),
    ]
    return pl.pallas_call(
        _node_mlp_body,
        grid=(grid,),
        in_specs=pspec + pspec + [
            pl.BlockSpec((_NBLK, _D), lambda i: (i, 0)),
            pl.BlockSpec((1, 1, _NBLK), lambda i: (i, 0, 0)),
            pl.BlockSpec((_D, _D), lambda i: (0, 0)),
            pl.BlockSpec((1, _D), lambda i: (0, 0)),
            pl.BlockSpec((_D, _D), lambda i: (0, 0)),
            pl.BlockSpec((1, _D), lambda i: (0, 0)),
            pl.BlockSpec(memory_space=pltpu.SMEM),
        ],
        out_specs=[
            pl.BlockSpec((_NBLK, _D), lambda i: (i, 0)),
            pl.BlockSpec((128, 128), lambda i: (0, 0)),
        ],
        out_shape=[
            jax.ShapeDtypeStruct((_N, _D), jnp.float32),
            jax.ShapeDtypeStruct((128, 128), jnp.float32),
        ],
    )(parts_a, parts_a, parts_a, parts_a, parts_b, parts_b, parts_b,
      parts_b, x, n2g3, w1, b1_row, w2, b2_row, eps)


def _final_body(h_ref, x_ref, st_ref, n2g_ref, w_ref, b_ref, o_ref):
    st = st_ref[...]
    cnt = st[:, 0:1]
    s1 = st[:, 1:2]
    s2 = st[:, 2:3]
    norm = jnp.maximum(cnt, 1.0) * float(_D)
    mean = s1 / norm
    var = s2 / norm - mean * mean
    inv = lax.rsqrt(var + 1e-5)

    n2g = n2g_ref[0, 0, :]
    ohn = (n2g[:, None]
           == lax.broadcasted_iota(jnp.int32, (_NBLK, 128), 1)).astype(jnp.float32)
    rmean = jnp.dot(ohn, mean, preferred_element_type=jnp.float32)
    rinv = jnp.dot(ohn, inv, preferred_element_type=jnp.float32)

    t = (h_ref[...] - rmean) * rinv * w_ref[...] + b_ref[...]
    o_ref[...] = jnp.maximum((t + x_ref[...]) * 0.5, 0.0)


def _final(h, x, st, n2g3, w_row, b_row):
    grid = _N // _NBLK
    return pl.pallas_call(
        _final_body,
        grid=(grid,),
        in_specs=[
            pl.BlockSpec((_NBLK, _D), lambda i: (i, 0)),
            pl.BlockSpec((_NBLK, _D), lambda i: (i, 0)),
            pl.BlockSpec((128, 128), lambda i: (0, 0)),
            pl.BlockSpec((1, 1, _NBLK), lambda i: (i, 0, 0)),
            pl.BlockSpec((1, _D), lambda i: (0, 0)),
            pl.BlockSpec((1, _D), lambda i: (0, 0)),
        ],
        out_specs=pl.BlockSpec((_NBLK, _D), lambda i: (i, 0)),
        out_shape=jax.ShapeDtypeStruct((_N, _D), jnp.float32),
    )(h, x, st, n2g3, w_row, b_row)


def kernel(x, edge_index, edge_attr, node2graph,
           c1_edge_W, c1_edge_b, c1_W1, c1_b1, c1_W2, c1_b2, c1_eps,
           c2_edge_W, c2_edge_b, c2_W1, c2_b1, c2_W2, c2_b2, c2_eps,
           gn1_w, gn1_b, gn2_w, gn2_b):
    src = edge_index[0].astype(jnp.int32)
    dst = edge_index[1].astype(jnp.int32)

    pad_e = _EP - _E
    psrc = jnp.arange(pad_e, dtype=jnp.int32) % _N
    src_p = jnp.concatenate([src, psrc])
    trash = _N + (jnp.arange(pad_e, dtype=jnp.int32) % (_NP - _N))
    dst_p = jnp.concatenate([dst, trash])
    srcs = src_p.reshape(2, _NW, _NCH, _CH)
    dsts = dst_p.reshape(2, _NW, _NCH, _CH)

    x0 = x[:, :128]
    x1 = x[:, 128:]

    eb_row = c2_edge_b.reshape(1, _D)
    ea0, ea1 = _edge_mlp(edge_attr, c2_edge_W, eb_row, 0)
    eb0, eb1 = _edge_mlp(edge_attr, c2_edge_W, eb_row, 1)
    parts_a = _sc_edge(x0, x1, ea0, ea1, srcs[0], dsts[0])
    parts_b = _sc_edge(x0, x1, eb0, eb1, srcs[1], dsts[1])

    n2g3 = node2graph.astype(jnp.int32).reshape(_N // _NBLK, 1, _NBLK)

    h, st = _node_mlp(parts_a, parts_b, x, n2g3, c2_W1,
                      c2_b1.reshape(1, _D), c2_W2, c2_b2.reshape(1, _D),
                      c2_eps)
    out = _final(h, x, st, n2g3, gn2_w.reshape(1, _D),
                 gn2_b.reshape(1, _D))
    return out
